# Initial kernel scaffold; baseline (speedup 1.0000x reference)
import jax
import jax.numpy as jnp
from jax.experimental import pallas as pl


def kernel(input, weight):
    raise NotImplementedError("write your pallas kernel here")


# speedup vs baseline: 2.9436x; 2.9436x over previous
import functools

import jax
import jax.numpy as jnp
from jax import lax
from jax.experimental import pallas as pl
from jax.experimental.pallas import tpu as pltpu
from jax.experimental.pallas import tpu_sc as plsc

B = 16384
L = 50
D = 32
NC, NS = 2, 16
NW = NC * NS
BPW = B // NW
C = 16
RPC = C * L
NCHUNK = BPW // C

_mesh = plsc.VectorSubcoreMesh(core_axis_name="c", subcore_axis_name="s")


@functools.partial(
    pl.kernel,
    out_type=jax.ShapeDtypeStruct((B, D), jnp.float32),
    mesh=_mesh,
    compiler_params=pltpu.CompilerParams(use_tc_tiling_on_sc=False),
    scratch_types=[
        pltpu.VMEM((BPW * L,), jnp.int32),
        pltpu.VMEM((RPC, D), jnp.float32),
        pltpu.VMEM((RPC, D), jnp.float32),
        pltpu.VMEM((C, D), jnp.float32),
        pltpu.VMEM((C, D), jnp.float32),
        pltpu.SemaphoreType.DMA,
        pltpu.SemaphoreType.DMA,
    ],
)
def _emb_bag(idx_hbm, w_hbm, out_hbm, idx_v, rows0, rows1, ob0, ob1, s0, s1):
    wid = lax.axis_index("s") * NC + lax.axis_index("c")
    base = wid * BPW
    pltpu.sync_copy(idx_hbm.at[pl.ds(base * L, BPW * L)], idx_v)

    rows = (rows0, rows1)
    obs = (ob0, ob1)
    sems = (s0, s1)

    for b in range(2):
        pltpu.async_copy(
            w_hbm.at[idx_v.at[pl.ds(b * RPC, RPC)]], rows[b], sems[b])

    def outer(g2, carry):
        for b in range(2):
            g = g2 * 2 + b
            pltpu.make_async_copy(
                w_hbm.at[idx_v.at[pl.ds(g * RPC, RPC)]], rows[b], sems[b]
            ).wait()

            def inner(c, _, b=b):
                r0 = c * L
                rbuf = rows[b]
                a0 = rbuf[r0, 0:16]
                a1 = rbuf[r0, 16:32]
                c0 = rbuf[r0 + 1, 0:16]
                c1 = rbuf[r0 + 1, 16:32]
                for l in range(2, L, 2):
                    a0 = a0 + rbuf[r0 + l, 0:16]
                    a1 = a1 + rbuf[r0 + l, 16:32]
                    c0 = c0 + rbuf[r0 + l + 1, 0:16]
                    c1 = c1 + rbuf[r0 + l + 1, 16:32]
                obs[b][c, 0:16] = a0 + c0
                obs[b][c, 16:32] = a1 + c1
                return 0

            lax.fori_loop(0, C, inner, 0)
            pltpu.sync_copy(obs[b], out_hbm.at[pl.ds(base + g * C, C)])

            @pl.when(g + 2 < NCHUNK)
            def _(b=b, g=g):
                pltpu.async_copy(
                    w_hbm.at[idx_v.at[pl.ds((g + 2) * RPC, RPC)]],
                    rows[b], sems[b])
        return carry

    lax.fori_loop(0, NCHUNK // 2, outer, 0)


def kernel(input, weight):
    idx = input.astype(jnp.int32).T.reshape(-1)
    return _emb_bag(idx, weight)
